# baseline (speedup 1.0000x reference)
import functools

import jax
import jax.numpy as jnp
from jax import lax
from jax.experimental import pallas as pl
from jax.experimental.pallas import tpu as pltpu
from jax.experimental.pallas import tpu_sc as plsc

_NC = 2
_NS = 16
_NW = _NC * _NS


def _make_gather(v, d, hist, batch):
    bpw = batch // _NW
    c = bpw // 2
    assert bpw * _NW == batch and 2 * c == bpw
    mesh = plsc.VectorSubcoreMesh(core_axis_name="c", subcore_axis_name="s",
                                  num_cores=_NC, num_subcores=_NS)

    @functools.partial(
        pl.kernel,
        mesh=mesh,
        out_type=jax.ShapeDtypeStruct((hist, batch, d), jnp.float32),
        compiler_params=pltpu.CompilerParams(use_tc_tiling_on_sc=False),
        scratch_types=[
            pltpu.VMEM((c,), jnp.int32),
            pltpu.VMEM((c,), jnp.int32),
            pltpu.VMEM((c, 2 * d), jnp.float32),
            pltpu.VMEM((c, 2 * d), jnp.float32),
            pltpu.SemaphoreType.DMA,
            pltpu.SemaphoreType.DMA,
            pltpu.SemaphoreType.DMA,
            pltpu.SemaphoreType.DMA,
        ],
    )
    def gather_kernel(tab_hbm, xt_hbm, out_hbm, i0, i1, r0, r1,
                      g0, g1, o0, o1):
        idx_v = [i0, i1]
        rows = [r0, r1]
        gsem = [g0, g1]
        osem = [o0, o1]
        w = lax.axis_index("s") * _NC + lax.axis_index("c")
        b_base = w * bpw

        def prep(h, bf, slot):
            pltpu.sync_copy(xt_hbm.at[h, pl.ds(b_base + bf * c, c)],
                            idx_v[slot])

        def fire_gather(slot):
            pltpu.async_copy(tab_hbm.at[idx_v[slot]], rows[slot],
                             gsem[slot])

        def wait_gather(slot):
            pltpu.make_async_copy(tab_hbm.at[idx_v[slot]], rows[slot],
                                  gsem[slot]).wait()

        def fire_out(h, bf, slot):
            pltpu.async_copy(rows[slot].at[:, pl.ds(0, d)],
                             out_hbm.at[h, pl.ds(b_base + bf * c, c), :],
                             osem[slot])

        def wait_out(h, bf, slot):
            pltpu.make_async_copy(rows[slot].at[:, pl.ds(0, d)],
                                  out_hbm.at[h, pl.ds(b_base + bf * c, c),
                                             :],
                                  osem[slot]).wait()

        prep(0, 0, 0)
        fire_gather(0)

        @pl.loop(0, hist)
        def _(h):
            prep(h, 1, 1)

            @pl.when(h >= 1)
            def _():
                wait_out(h - 1, 1, 1)

            fire_gather(1)
            wait_gather(0)
            fire_out(h, 0, 0)

            @pl.when(h + 1 < hist)
            def _():
                prep(h + 1, 0, 0)
                wait_out(h, 0, 0)
                fire_gather(0)

            wait_gather(1)
            fire_out(h, 1, 1)

        wait_out(hist - 1, 0, 0)
        wait_out(hist - 1, 1, 1)

    return gather_kernel


@jax.jit
def kernel(x, table):
    batch, hist = x.shape
    vocab, dim = table.shape
    xt = x.T
    tp = jnp.pad(table, ((0, 0), (0, dim)))
    out_hm = _make_gather(vocab, dim, hist, batch)(tp, xt)
    return jnp.transpose(out_hm, (1, 0, 2))


# speedup vs baseline: 1.0458x; 1.0458x over previous
import functools

import jax
import jax.numpy as jnp
from jax import lax
from jax.experimental import pallas as pl
from jax.experimental.pallas import tpu as pltpu
from jax.experimental.pallas import tpu_sc as plsc

_NC = 2
_NS = 16
_NW = _NC * _NS


def _make_gather(v, d, hist, batch):
    bpw = batch // _NW
    assert bpw * _NW == batch and hist % 2 == 0
    mesh = plsc.VectorSubcoreMesh(core_axis_name="c", subcore_axis_name="s",
                                  num_cores=_NC, num_subcores=_NS)

    @functools.partial(
        pl.kernel,
        mesh=mesh,
        out_type=jax.ShapeDtypeStruct((hist, batch, d), jnp.float32),
        compiler_params=pltpu.CompilerParams(use_tc_tiling_on_sc=False),
        scratch_types=[
            pltpu.VMEM((bpw,), jnp.int32),
            pltpu.VMEM((bpw,), jnp.int32),
            pltpu.VMEM((bpw, d), jnp.float32),
            pltpu.VMEM((bpw, d), jnp.float32),
            pltpu.SemaphoreType.DMA,
            pltpu.SemaphoreType.DMA,
            pltpu.SemaphoreType.DMA,
            pltpu.SemaphoreType.DMA,
        ],
    )
    def gather_kernel(tab_hbm, xt_hbm, out_hbm, i0, i1, r0, r1,
                      g0, g1, o0, o1):
        idx_v = [i0, i1]
        rows = [r0, r1]
        gsem = [g0, g1]
        osem = [o0, o1]
        w = lax.axis_index("s") * _NC + lax.axis_index("c")
        b_base = w * bpw

        def prep(h, slot):
            pltpu.sync_copy(xt_hbm.at[h, pl.ds(b_base, bpw)], idx_v[slot])

        def fire_gather(slot):
            pltpu.async_copy(tab_hbm.at[idx_v[slot]], rows[slot],
                             gsem[slot])

        def wait_gather(slot):
            pltpu.make_async_copy(tab_hbm.at[idx_v[slot]], rows[slot],
                                  gsem[slot]).wait()

        def fire_out(h, slot):
            pltpu.async_copy(rows[slot],
                             out_hbm.at[h, pl.ds(b_base, bpw), :],
                             osem[slot])

        def wait_out(h, slot):
            pltpu.make_async_copy(rows[slot],
                                  out_hbm.at[h, pl.ds(b_base, bpw), :],
                                  osem[slot]).wait()

        prep(0, 0)
        fire_gather(0)

        @pl.loop(0, hist // 2)
        def _(hh):
            for p in (0, 1):
                h = 2 * hh + p
                s = p
                o = 1 - p

                @pl.when(h + 1 < hist)
                def _(h=h, s=s, o=o):
                    prep(h + 1, o)

                    @pl.when(h >= 1)
                    def _():
                        wait_out(h - 1, o)

                    fire_gather(o)

                wait_gather(s)
                fire_out(h, s)

        wait_out(hist - 2, 0)
        wait_out(hist - 1, 1)

    return gather_kernel


@jax.jit
def kernel(x, table):
    batch, hist = x.shape
    vocab, dim = table.shape
    xt = x.T
    out_hm = _make_gather(vocab, dim, hist, batch)(table, xt)
    return jnp.transpose(out_hm, (1, 0, 2))
